# baseline (speedup 1.0000x reference)
import functools

import jax
import jax.numpy as jnp
from jax import lax
from jax.experimental import pallas as pl
from jax.experimental.pallas import tpu as pltpu
from jax.experimental.pallas import tpu_sc as plsc

NC = 2
NS = 16
LC = 16
CH = 80
ZCH = 80
ZR = 80


def _seg_kernel(N, EL, D, split, offn):
    epw = EL // NC // NS if split else EL // NS
    nchunks = epw // CH
    nzch = N // ZCH
    nzit = -(-nzch // NS)

    mesh = plsc.VectorSubcoreMesh(core_axis_name="c", subcore_axis_name="s")

    @functools.partial(
        pl.kernel,
        out_type=jax.ShapeDtypeStruct((NC * N, D), jnp.float32),
        mesh=mesh,
        scratch_types=[
            pltpu.VMEM((CH,), jnp.int32),
            pltpu.VMEM((CH,), jnp.int32),
            pltpu.VMEM((CH,), jnp.int32),
            pltpu.VMEM((CH,), jnp.int32),
            pltpu.VMEM((CH, D), jnp.float32),
            pltpu.VMEM((CH, D), jnp.float32),
            pltpu.VMEM((ZR, D), jnp.float32),
            pltpu.VMEM_SHARED((N, D), jnp.float32),
            pltpu.SemaphoreType.DMA,
            pltpu.SemaphoreType.DMA,
            pltpu.SemaphoreType.DMA,
            pltpu.SemaphoreType.DMA,
            pltpu.SemaphoreType.DMA,
            pltpu.SemaphoreType.DMA,
        ],
    )
    def kseg(idx_hbm, dst_hbm, table_hbm, acc_out,
             src0, src1, dst0, dst1, rows0, rows1, z_wide, s_sh,
             ss0, ss1, sd0, sd1, sg0, sg1):
        cid = lax.axis_index("c")
        sid = lax.axis_index("s")

        zf = jnp.zeros((LC,), jnp.float32)

        def initz(i, _):
            def initcol(j, _):
                z_wide[i, pl.ds(j * LC, LC)] = zf
                return 0
            lax.fori_loop(0, D // LC, initcol, 0)
            return 0
        lax.fori_loop(0, ZR, initz, 0)

        def zloop(k, _):
            idx = k * NS + sid

            @pl.when(idx < nzch)
            def _():
                pltpu.sync_copy(z_wide.at[pl.ds(0, ZCH)],
                                s_sh.at[pl.ds(idx * ZCH, ZCH)])
            return 0
        lax.fori_loop(0, nzit, zloop, 0)
        plsc.subcore_barrier()

        if split:
            ebase = (cid * NS + sid) * epw
            off = 0
        else:
            ebase = sid * epw
            off = cid * offn
        sets = ((src0, dst0, ss0, sd0, rows0, sg0),
                (src1, dst1, ss1, sd1, rows1, sg1))

        def start_idx(i, s):
            base = ebase + i * CH
            pltpu.async_copy(idx_hbm.at[pl.ds(base, CH)], s[0], s[2])
            pltpu.async_copy(dst_hbm.at[pl.ds(base, CH)], s[1], s[3])

        def wait_idx(s):
            pltpu.make_async_copy(idx_hbm.at[pl.ds(0, CH)], s[0], s[2]).wait()
            pltpu.make_async_copy(dst_hbm.at[pl.ds(0, CH)], s[1], s[3]).wait()

        def wait_scatter(s):
            pltpu.make_async_copy(table_hbm.at[s[0]], s[4], s[5]).wait()
            pltpu.sync_copy(s[4], s_sh.at[s[1]], add=True)

        def step(i, cur, oth):
            wait_idx(cur)
            if not split:
                def addoff(g, _):
                    cur[0][pl.ds(g * LC, LC)] = cur[0][pl.ds(g * LC, LC)] + off
                    return 0
                lax.fori_loop(0, CH // LC, addoff, 0)
            pltpu.async_copy(table_hbm.at[cur[0]], cur[4], cur[5])

            @pl.when(i > 0)
            def _():
                wait_scatter(oth)

            @pl.when(i + 1 < nchunks)
            def _():
                start_idx(i + 1, oth)

        start_idx(0, sets[0])

        def pair(j, _):
            step(2 * j, sets[0], sets[1])
            step(2 * j + 1, sets[1], sets[0])
            return 0
        lax.fori_loop(0, nchunks // 2, pair, 0)
        if nchunks % 2:
            step(nchunks - 1, sets[0], sets[1])
            wait_scatter(sets[0])
        else:
            wait_scatter(sets[1])
        plsc.subcore_barrier()

        def outloop(k, _):
            idx = k * NS + sid

            @pl.when(idx < nzch)
            def _():
                rb = idx * ZCH
                pltpu.sync_copy(s_sh.at[pl.ds(rb, ZCH)],
                                acc_out.at[pl.ds(cid * N + rb, ZCH)])
            return 0
        lax.fori_loop(0, nzit, outloop, 0)

    return kseg


def _tcnt_kernel(N, E, DE, D):
    epw = E // NS
    nchunks = epw // CH
    nzch = N // ZCH
    nzit = -(-nzch // NS)

    mesh = plsc.VectorSubcoreMesh(core_axis_name="c", subcore_axis_name="s")

    @functools.partial(
        pl.kernel,
        out_type=jax.ShapeDtypeStruct((NC * N, D), jnp.float32),
        mesh=mesh,
        scratch_types=[
            pltpu.VMEM((CH,), jnp.int32),
            pltpu.VMEM((CH,), jnp.int32),
            pltpu.VMEM((CH * DE,), jnp.float32),
            pltpu.VMEM((CH * DE,), jnp.float32),
            pltpu.VMEM((CH, D), jnp.float32),
            pltpu.VMEM((CH, D), jnp.float32),
            pltpu.VMEM((ZR, D), jnp.float32),
            pltpu.VMEM_SHARED((N, D), jnp.float32),
            pltpu.SemaphoreType.DMA,
            pltpu.SemaphoreType.DMA,
        ],
    )
    def kt(dst_hbm, dstn_hbm, efeats_hbm, acc_out,
           dst0, dst1, e0, e1, wide0, wide1, z_wide, acc_sh,
           se0, se1):
        cid = lax.axis_index("c")
        sid = lax.axis_index("s")

        zf = jnp.zeros((LC,), jnp.float32)
        of = jnp.ones((LC,), jnp.float32)

        def initz(i, _):
            def initcol(j, _):
                z_wide[i, pl.ds(j * LC, LC)] = zf
                return 0
            lax.fori_loop(0, D // LC, initcol, 0)
            return 0
        lax.fori_loop(0, ZR, initz, 0)

        def initwide(wide_v):
            def initrow(i, _):
                def initcol(j, _):
                    wide_v[i, pl.ds(j * LC, LC)] = zf
                    return 0
                lax.fori_loop(0, D // LC, initcol, 0)
                wide_v[i, pl.ds(DE, LC)] = of
                return 0
            lax.fori_loop(0, CH, initrow, 0)
        initwide(wide0)
        initwide(wide1)

        def zloop(k, _):
            idx = k * NS + sid

            @pl.when(idx < nzch)
            def _():
                pltpu.sync_copy(z_wide.at[pl.ds(0, ZCH)],
                                acc_sh.at[pl.ds(idx * ZCH, ZCH)])
            return 0
        lax.fori_loop(0, nzit, zloop, 0)
        plsc.subcore_barrier()

        lbase = sid * epw

        sets = ((dst0, e0, se0, wide0),
                (dst1, e1, se1, wide1))

        def start(i, s):
            pltpu.async_copy(
                efeats_hbm.at[pl.ds((lbase + i * CH) * DE, CH * DE)],
                s[1], s[2])

        def step(i, cur, oth):
            @pl.when(cid == 0)
            def _():
                pltpu.sync_copy(dst_hbm.at[pl.ds(lbase + i * CH, CH)],
                                cur[0])

            @pl.when(cid == 1)
            def _():
                pltpu.sync_copy(dstn_hbm.at[pl.ds(lbase + i * CH, CH)],
                                cur[0])
            pltpu.make_async_copy(efeats_hbm.at[pl.ds(0, CH * DE)],
                                  cur[1], cur[2]).wait()

            @pl.when(i + 1 < nchunks)
            def _():
                start(i + 1, oth)

            def pack(r, _):
                cur[3][r, pl.ds(0, LC)] = cur[1][pl.ds(r * DE, LC)]
                return 0
            lax.fori_loop(0, CH, pack, 0)
            pltpu.sync_copy(cur[3], acc_sh.at[cur[0]], add=True)

        start(0, sets[0])

        def pair(j, _):
            step(2 * j, sets[0], sets[1])
            step(2 * j + 1, sets[1], sets[0])
            return 0
        lax.fori_loop(0, nchunks // 2, pair, 0)
        if nchunks % 2:
            step(nchunks - 1, sets[0], sets[1])
        plsc.subcore_barrier()

        def outloop(k, _):
            idx = k * NS + sid

            @pl.when(idx < nzch)
            def _():
                rb = idx * ZCH
                pltpu.sync_copy(acc_sh.at[pl.ds(rb, ZCH)],
                                acc_out.at[pl.ds(cid * N + rb, ZCH)])
            return 0
        lax.fori_loop(0, nzit, outloop, 0)

    return kt


def _tc_layer1(N, DIN, DE, DOUT, D, BN):
    ng = N // BN

    def body(nf, s1, acc, wmh, wme_aug, wah, wan, ba, out):
        s = s1[0] + s1[1]
        afp = acc[0]
        afn = acc[1]
        cm = jnp.maximum(afp[:, DE:DE + 1], 1.0)
        sh = jnp.dot(s, wmh[...], preferred_element_type=jnp.float32)
        hnp = (sh + jnp.dot(afp, wme_aug[...],
                            preferred_element_type=jnp.float32)) / cm
        hnn = (sh + jnp.dot(afn, wme_aug[...],
                            preferred_element_type=jnp.float32)) / cm
        base = jnp.dot(nf[...], wah[...],
                       preferred_element_type=jnp.float32) + ba[...]
        out[0] = jax.nn.relu(
            base + jnp.dot(hnp, wan[...], preferred_element_type=jnp.float32))
        out[1] = jax.nn.relu(
            base + jnp.dot(hnn, wan[...], preferred_element_type=jnp.float32))

    full = lambda i: (0, 0)
    return pl.pallas_call(
        body,
        grid=(ng,),
        in_specs=[
            pl.BlockSpec((BN, DIN), lambda i: (i, 0)),
            pl.BlockSpec((NC, BN, DIN), lambda i: (0, i, 0)),
            pl.BlockSpec((NC, BN, D), lambda i: (0, i, 0)),
            pl.BlockSpec((DIN, DOUT), full),
            pl.BlockSpec((D, DOUT), full),
            pl.BlockSpec((DIN, DOUT), full),
            pl.BlockSpec((DOUT, DOUT), full),
            pl.BlockSpec((1, DOUT), full),
        ],
        out_specs=pl.BlockSpec((2, BN, DOUT), lambda i: (0, i, 0)),
        out_shape=jax.ShapeDtypeStruct((2, N, DOUT), jnp.float32),
    )


def _tc_layer2_loss(N, DE, DOUT, D, BN):
    ng = N // BN
    inv = 1.0 / (N * DOUT)

    def body(h1, s2, acc, wmh, wme_aug, wah, wan, ba, out):
        i = pl.program_id(0)
        afp = acc[0]
        afn = acc[1]
        cm = jnp.maximum(afp[:, DE:DE + 1], 1.0)

        def head(s, t):
            return (jnp.dot(s, wmh[...], preferred_element_type=jnp.float32)
                    + jnp.dot(t, wme_aug[...],
                              preferred_element_type=jnp.float32)) / cm

        hnp = head(s2[0], afp)
        hnn = head(s2[1], afn)
        h2p = jax.nn.relu(
            jnp.dot(h1[0], wah[...], preferred_element_type=jnp.float32)
            + jnp.dot(hnp, wan[...], preferred_element_type=jnp.float32)
            + ba[...])
        h2n = jax.nn.relu(
            jnp.dot(h1[1], wah[...], preferred_element_type=jnp.float32)
            + jnp.dot(hnn, wan[...], preferred_element_type=jnp.float32)
            + ba[...])
        lp = jnp.maximum(h2p, 0.0) - h2p + jnp.log1p(jnp.exp(-jnp.abs(h2p)))
        ln = jnp.maximum(h2n, 0.0) + jnp.log1p(jnp.exp(-jnp.abs(h2n)))
        part = jnp.sum(lp) + jnp.sum(ln)

        @pl.when(i == 0)
        def _():
            out[...] = jnp.zeros((1, 1), jnp.float32)

        out[...] += jnp.full((1, 1), part, jnp.float32)

        @pl.when(i == ng - 1)
        def _():
            out[...] = out[...] * inv

    full = lambda i: (0, 0)
    return pl.pallas_call(
        body,
        grid=(ng,),
        in_specs=[
            pl.BlockSpec((2, BN, DOUT), lambda i: (0, i, 0)),
            pl.BlockSpec((2, BN, DOUT), lambda i: (0, i, 0)),
            pl.BlockSpec((NC, BN, D), lambda i: (0, i, 0)),
            pl.BlockSpec((DOUT, DOUT), full),
            pl.BlockSpec((D, DOUT), full),
            pl.BlockSpec((DOUT, DOUT), full),
            pl.BlockSpec((DOUT, DOUT), full),
            pl.BlockSpec((1, DOUT), full),
        ],
        out_specs=pl.BlockSpec((1, 1), full),
        out_shape=jax.ShapeDtypeStruct((1, 1), jnp.float32),
    )


def kernel(nfeats, edge_index, efeats, Wmsg1, bmsg1, Wapply1, bapply1,
           Wedge1, bedge1, Wmsg2, bmsg2, Wapply2, bapply2, Wedge2, bedge2):
    N, DIN = nfeats.shape
    E, DE = efeats.shape
    DOUT = Wapply1.shape[1]
    BN = 1000

    src = edge_index[0]
    dst = edge_index[1]
    perm = jax.random.permutation(jax.random.key(1), E).astype(jnp.int32)
    inv_perm = jnp.argsort(perm)
    dstn = jnp.take(dst, inv_perm)

    D = 128

    s1p = _seg_kernel(N, E, DIN, True, 0)(src, dst, nfeats).reshape(NC, N, DIN)

    acc = _tcnt_kernel(N, E, DE, D)(
        dst, dstn, efeats.reshape(-1)).reshape(NC, N, D)

    def aug(wme, bm):
        return jnp.concatenate(
            [wme, bm.reshape(1, DOUT),
             jnp.zeros((D - DE - 1, DOUT), jnp.float32)], axis=0)

    h1 = _tc_layer1(N, DIN, DE, DOUT, D, BN)(
        nfeats, s1p, acc,
        Wmsg1[:DIN], aug(Wmsg1[DIN:], bmsg1),
        Wapply1[:DIN], Wapply1[DIN:], bapply1.reshape(1, DOUT))

    s2 = _seg_kernel(N, E, DOUT, False, N)(
        src, dst, h1.reshape(2 * N, DOUT)).reshape(NC, N, DOUT)

    loss = _tc_layer2_loss(N, DE, DOUT, D, BN)(
        h1, s2, acc,
        Wmsg2[:DOUT], aug(Wmsg2[DOUT:], bmsg2),
        Wapply2[:DOUT], Wapply2[DOUT:], bapply2.reshape(1, DOUT))
    return loss[0, 0]


# speedup vs baseline: 1.0921x; 1.0921x over previous
import functools

import jax
import jax.numpy as jnp
from jax import lax
from jax.experimental import pallas as pl
from jax.experimental.pallas import tpu as pltpu
from jax.experimental.pallas import tpu_sc as plsc

NC = 2
NS = 16
LC = 16
CH = 80
ZCH = 80
ZR = 80


def _seg_kernel(N, EL, D, split, offn):
    epw = EL // NC // NS if split else EL // NS
    nchunks = epw // CH
    nzch = N // ZCH
    nzit = -(-nzch // NS)

    mesh = plsc.VectorSubcoreMesh(core_axis_name="c", subcore_axis_name="s")

    @functools.partial(
        pl.kernel,
        out_type=jax.ShapeDtypeStruct((NC * N, D), jnp.float32),
        mesh=mesh,
        scratch_types=[
            pltpu.VMEM((CH,), jnp.int32),
            pltpu.VMEM((CH,), jnp.int32),
            pltpu.VMEM((CH,), jnp.int32),
            pltpu.VMEM((CH,), jnp.int32),
            pltpu.VMEM((CH,), jnp.int32),
            pltpu.VMEM((CH,), jnp.int32),
            pltpu.VMEM((CH, D), jnp.float32),
            pltpu.VMEM((CH, D), jnp.float32),
            pltpu.VMEM((ZR, D), jnp.float32),
            pltpu.VMEM_SHARED((N, D), jnp.float32),
            pltpu.SemaphoreType.DMA,
            pltpu.SemaphoreType.DMA,
            pltpu.SemaphoreType.DMA,
            pltpu.SemaphoreType.DMA,
            pltpu.SemaphoreType.DMA,
            pltpu.SemaphoreType.DMA,
            pltpu.SemaphoreType.DMA,
            pltpu.SemaphoreType.DMA,
        ],
    )
    def kseg(idx_hbm, dst_hbm, table_hbm, acc_out,
             src0, src1, dst0, dst1, dsc0, dsc1, rows0, rows1, z_wide, s_sh,
             ss0, ss1, sd0, sd1, sg0, sg1, sc0, sc1):
        cid = lax.axis_index("c")
        sid = lax.axis_index("s")

        zf = jnp.zeros((LC,), jnp.float32)

        def initz(i, _):
            def initcol(j, _):
                z_wide[i, pl.ds(j * LC, LC)] = zf
                return 0
            lax.fori_loop(0, D // LC, initcol, 0)
            return 0
        lax.fori_loop(0, ZR, initz, 0)

        def zloop(k, _):
            idx = k * NS + sid

            @pl.when(idx < nzch)
            def _():
                pltpu.sync_copy(z_wide.at[pl.ds(0, ZCH)],
                                s_sh.at[pl.ds(idx * ZCH, ZCH)])
            return 0
        lax.fori_loop(0, nzit, zloop, 0)
        plsc.subcore_barrier()

        if split:
            ebase = (cid * NS + sid) * epw
            off = 0
        else:
            ebase = sid * epw
            off = cid * offn
        sets = ((src0, dst0, ss0, sd0, rows0, sg0, dsc0, sc0),
                (src1, dst1, ss1, sd1, rows1, sg1, dsc1, sc1))

        def start_idx(i, s):
            base = ebase + i * CH
            pltpu.async_copy(idx_hbm.at[pl.ds(base, CH)], s[0], s[2])
            pltpu.async_copy(dst_hbm.at[pl.ds(base, CH)], s[1], s[3])

        def wait_idx(s):
            pltpu.make_async_copy(idx_hbm.at[pl.ds(0, CH)], s[0], s[2]).wait()
            pltpu.make_async_copy(dst_hbm.at[pl.ds(0, CH)], s[1], s[3]).wait()

        def start_scatter(s):
            def cp(g, _):
                s[6][pl.ds(g * LC, LC)] = s[1][pl.ds(g * LC, LC)]
                return 0
            lax.fori_loop(0, CH // LC, cp, 0)
            pltpu.async_copy(s[4], s_sh.at[s[6]], s[7], add=True)

        def wait_gather(s):
            pltpu.make_async_copy(table_hbm.at[s[0]], s[4], s[5]).wait()

        def wait_scatter(s):
            pltpu.make_async_copy(s[4], s_sh.at[s[6]], s[7]).wait()

        def step(i, cur, oth):
            wait_idx(cur)
            if not split:
                def addoff(g, _):
                    cur[0][pl.ds(g * LC, LC)] = cur[0][pl.ds(g * LC, LC)] + off
                    return 0
                lax.fori_loop(0, CH // LC, addoff, 0)

            @pl.when(i > 1)
            def _():
                wait_scatter(cur)
            pltpu.async_copy(table_hbm.at[cur[0]], cur[4], cur[5])

            @pl.when(i > 0)
            def _():
                wait_gather(oth)
                start_scatter(oth)

            @pl.when(i + 1 < nchunks)
            def _():
                start_idx(i + 1, oth)

        start_idx(0, sets[0])

        def pair(j, _):
            step(2 * j, sets[0], sets[1])
            step(2 * j + 1, sets[1], sets[0])
            return 0
        lax.fori_loop(0, nchunks // 2, pair, 0)
        if nchunks % 2:
            step(nchunks - 1, sets[0], sets[1])
        last = sets[(nchunks - 1) % 2]
        prev = sets[nchunks % 2]
        wait_gather(last)
        start_scatter(last)
        wait_scatter(prev)
        wait_scatter(last)
        plsc.subcore_barrier()

        def outloop(k, _):
            idx = k * NS + sid

            @pl.when(idx < nzch)
            def _():
                rb = idx * ZCH
                pltpu.sync_copy(s_sh.at[pl.ds(rb, ZCH)],
                                acc_out.at[pl.ds(cid * N + rb, ZCH)])
            return 0
        lax.fori_loop(0, nzit, outloop, 0)

    return kseg


def _tcnt_kernel(N, E, DE, D):
    epw = E // NS
    nchunks = epw // CH
    nzch = N // ZCH
    nzit = -(-nzch // NS)

    mesh = plsc.VectorSubcoreMesh(core_axis_name="c", subcore_axis_name="s")

    @functools.partial(
        pl.kernel,
        out_type=jax.ShapeDtypeStruct((NC * N, D), jnp.float32),
        mesh=mesh,
        scratch_types=[
            pltpu.VMEM((CH,), jnp.int32),
            pltpu.VMEM((CH,), jnp.int32),
            pltpu.VMEM((CH * DE,), jnp.float32),
            pltpu.VMEM((CH * DE,), jnp.float32),
            pltpu.VMEM((CH, D), jnp.float32),
            pltpu.VMEM((CH, D), jnp.float32),
            pltpu.VMEM((ZR, D), jnp.float32),
            pltpu.VMEM_SHARED((N, D), jnp.float32),
            pltpu.SemaphoreType.DMA,
            pltpu.SemaphoreType.DMA,
            pltpu.SemaphoreType.DMA,
            pltpu.SemaphoreType.DMA,
        ],
    )
    def kt(dst_hbm, dstn_hbm, efeats_hbm, acc_out,
           dst0, dst1, e0, e1, wide0, wide1, z_wide, acc_sh,
           se0, se1, sc0, sc1):
        cid = lax.axis_index("c")
        sid = lax.axis_index("s")

        zf = jnp.zeros((LC,), jnp.float32)
        of = jnp.ones((LC,), jnp.float32)

        def initz(i, _):
            def initcol(j, _):
                z_wide[i, pl.ds(j * LC, LC)] = zf
                return 0
            lax.fori_loop(0, D // LC, initcol, 0)
            return 0
        lax.fori_loop(0, ZR, initz, 0)

        def initwide(wide_v):
            def initrow(i, _):
                def initcol(j, _):
                    wide_v[i, pl.ds(j * LC, LC)] = zf
                    return 0
                lax.fori_loop(0, D // LC, initcol, 0)
                wide_v[i, pl.ds(DE, LC)] = of
                return 0
            lax.fori_loop(0, CH, initrow, 0)
        initwide(wide0)
        initwide(wide1)

        def zloop(k, _):
            idx = k * NS + sid

            @pl.when(idx < nzch)
            def _():
                pltpu.sync_copy(z_wide.at[pl.ds(0, ZCH)],
                                acc_sh.at[pl.ds(idx * ZCH, ZCH)])
            return 0
        lax.fori_loop(0, nzit, zloop, 0)
        plsc.subcore_barrier()

        lbase = sid * epw

        sets = ((dst0, e0, se0, wide0, sc0),
                (dst1, e1, se1, wide1, sc1))

        def start(i, s):
            pltpu.async_copy(
                efeats_hbm.at[pl.ds((lbase + i * CH) * DE, CH * DE)],
                s[1], s[2])

        def wait_scatter(s):
            pltpu.make_async_copy(s[3], acc_sh.at[s[0]], s[4]).wait()

        def step(i, cur, oth):
            @pl.when(i > 1)
            def _():
                wait_scatter(cur)

            @pl.when(cid == 0)
            def _():
                pltpu.sync_copy(dst_hbm.at[pl.ds(lbase + i * CH, CH)],
                                cur[0])

            @pl.when(cid == 1)
            def _():
                pltpu.sync_copy(dstn_hbm.at[pl.ds(lbase + i * CH, CH)],
                                cur[0])
            pltpu.make_async_copy(efeats_hbm.at[pl.ds(0, CH * DE)],
                                  cur[1], cur[2]).wait()

            @pl.when(i + 1 < nchunks)
            def _():
                start(i + 1, oth)

            def pack(r, _):
                cur[3][r, pl.ds(0, LC)] = cur[1][pl.ds(r * DE, LC)]
                return 0
            lax.fori_loop(0, CH, pack, 0)
            pltpu.async_copy(cur[3], acc_sh.at[cur[0]], cur[4], add=True)

        start(0, sets[0])

        def pair(j, _):
            step(2 * j, sets[0], sets[1])
            step(2 * j + 1, sets[1], sets[0])
            return 0
        lax.fori_loop(0, nchunks // 2, pair, 0)
        if nchunks % 2:
            step(nchunks - 1, sets[0], sets[1])
        wait_scatter(sets[nchunks % 2])
        wait_scatter(sets[(nchunks - 1) % 2])
        plsc.subcore_barrier()

        def outloop(k, _):
            idx = k * NS + sid

            @pl.when(idx < nzch)
            def _():
                rb = idx * ZCH
                pltpu.sync_copy(acc_sh.at[pl.ds(rb, ZCH)],
                                acc_out.at[pl.ds(cid * N + rb, ZCH)])
            return 0
        lax.fori_loop(0, nzit, outloop, 0)

    return kt


def _tc_layer1(N, DIN, DE, DOUT, D, BN):
    ng = N // BN

    def body(nf, s1, acc, wmh, wme_aug, wah, wan, ba, out):
        s = s1[0] + s1[1]
        afp = acc[0]
        afn = acc[1]
        cm = jnp.maximum(afp[:, DE:DE + 1], 1.0)
        sh = jnp.dot(s, wmh[...], preferred_element_type=jnp.float32)
        hnp = (sh + jnp.dot(afp, wme_aug[...],
                            preferred_element_type=jnp.float32)) / cm
        hnn = (sh + jnp.dot(afn, wme_aug[...],
                            preferred_element_type=jnp.float32)) / cm
        base = jnp.dot(nf[...], wah[...],
                       preferred_element_type=jnp.float32) + ba[...]
        out[0] = jax.nn.relu(
            base + jnp.dot(hnp, wan[...], preferred_element_type=jnp.float32))
        out[1] = jax.nn.relu(
            base + jnp.dot(hnn, wan[...], preferred_element_type=jnp.float32))

    full = lambda i: (0, 0)
    return pl.pallas_call(
        body,
        grid=(ng,),
        in_specs=[
            pl.BlockSpec((BN, DIN), lambda i: (i, 0)),
            pl.BlockSpec((NC, BN, DIN), lambda i: (0, i, 0)),
            pl.BlockSpec((NC, BN, D), lambda i: (0, i, 0)),
            pl.BlockSpec((DIN, DOUT), full),
            pl.BlockSpec((D, DOUT), full),
            pl.BlockSpec((DIN, DOUT), full),
            pl.BlockSpec((DOUT, DOUT), full),
            pl.BlockSpec((1, DOUT), full),
        ],
        out_specs=pl.BlockSpec((2, BN, DOUT), lambda i: (0, i, 0)),
        out_shape=jax.ShapeDtypeStruct((2, N, DOUT), jnp.float32),
    )


def _tc_layer2_loss(N, DE, DOUT, D, BN):
    ng = N // BN
    inv = 1.0 / (N * DOUT)

    def body(h1, s2, acc, wmh, wme_aug, wah, wan, ba, out):
        i = pl.program_id(0)
        afp = acc[0]
        afn = acc[1]
        cm = jnp.maximum(afp[:, DE:DE + 1], 1.0)

        def head(s, t):
            return (jnp.dot(s, wmh[...], preferred_element_type=jnp.float32)
                    + jnp.dot(t, wme_aug[...],
                              preferred_element_type=jnp.float32)) / cm

        hnp = head(s2[0], afp)
        hnn = head(s2[1], afn)
        h2p = jax.nn.relu(
            jnp.dot(h1[0], wah[...], preferred_element_type=jnp.float32)
            + jnp.dot(hnp, wan[...], preferred_element_type=jnp.float32)
            + ba[...])
        h2n = jax.nn.relu(
            jnp.dot(h1[1], wah[...], preferred_element_type=jnp.float32)
            + jnp.dot(hnn, wan[...], preferred_element_type=jnp.float32)
            + ba[...])
        lp = jnp.maximum(h2p, 0.0) - h2p + jnp.log1p(jnp.exp(-jnp.abs(h2p)))
        ln = jnp.maximum(h2n, 0.0) + jnp.log1p(jnp.exp(-jnp.abs(h2n)))
        part = jnp.sum(lp) + jnp.sum(ln)

        @pl.when(i == 0)
        def _():
            out[...] = jnp.zeros((1, 1), jnp.float32)

        out[...] += jnp.full((1, 1), part, jnp.float32)

        @pl.when(i == ng - 1)
        def _():
            out[...] = out[...] * inv

    full = lambda i: (0, 0)
    return pl.pallas_call(
        body,
        grid=(ng,),
        in_specs=[
            pl.BlockSpec((2, BN, DOUT), lambda i: (0, i, 0)),
            pl.BlockSpec((2, BN, DOUT), lambda i: (0, i, 0)),
            pl.BlockSpec((NC, BN, D), lambda i: (0, i, 0)),
            pl.BlockSpec((DOUT, DOUT), full),
            pl.BlockSpec((D, DOUT), full),
            pl.BlockSpec((DOUT, DOUT), full),
            pl.BlockSpec((DOUT, DOUT), full),
            pl.BlockSpec((1, DOUT), full),
        ],
        out_specs=pl.BlockSpec((1, 1), full),
        out_shape=jax.ShapeDtypeStruct((1, 1), jnp.float32),
    )


def kernel(nfeats, edge_index, efeats, Wmsg1, bmsg1, Wapply1, bapply1,
           Wedge1, bedge1, Wmsg2, bmsg2, Wapply2, bapply2, Wedge2, bedge2):
    N, DIN = nfeats.shape
    E, DE = efeats.shape
    DOUT = Wapply1.shape[1]
    BN = 1000

    src = edge_index[0]
    dst = edge_index[1]
    perm = jax.random.permutation(jax.random.key(1), E).astype(jnp.int32)
    inv_perm = jnp.argsort(perm)
    dstn = jnp.take(dst, inv_perm)

    D = 128

    s1p = _seg_kernel(N, E, DIN, True, 0)(src, dst, nfeats).reshape(NC, N, DIN)

    acc = _tcnt_kernel(N, E, DE, D)(
        dst, dstn, efeats.reshape(-1)).reshape(NC, N, D)

    def aug(wme, bm):
        return jnp.concatenate(
            [wme, bm.reshape(1, DOUT),
             jnp.zeros((D - DE - 1, DOUT), jnp.float32)], axis=0)

    h1 = _tc_layer1(N, DIN, DE, DOUT, D, BN)(
        nfeats, s1p, acc,
        Wmsg1[:DIN], aug(Wmsg1[DIN:], bmsg1),
        Wapply1[:DIN], Wapply1[DIN:], bapply1.reshape(1, DOUT))

    s2 = _seg_kernel(N, E, DOUT, False, N)(
        src, dst, h1.reshape(2 * N, DOUT)).reshape(NC, N, DOUT)

    loss = _tc_layer2_loss(N, DE, DOUT, D, BN)(
        h1, s2, acc,
        Wmsg2[:DOUT], aug(Wmsg2[DOUT:], bmsg2),
        Wapply2[:DOUT], Wapply2[DOUT:], bapply2.reshape(1, DOUT))
    return loss[0, 0]
